# Initial kernel scaffold; baseline (speedup 1.0000x reference)
import jax
import jax.numpy as jnp
from jax.experimental import pallas as pl


def kernel(word_idxs, char_idxs, word_table, char_table, W_conv, b_conv, Wt0, bt0, Wg0, bg0, Wt1, bt1, Wg1, bg1):
    raise NotImplementedError("write your pallas kernel here")


# speedup vs baseline: 7.3911x; 7.3911x over previous
import functools

import jax
import jax.numpy as jnp
from jax import lax
from jax.experimental import pallas as pl
from jax.experimental.pallas import tpu as pltpu
from jax.experimental.pallas import tpu_sc as plsc

B, S, L = 64, 400, 16
VW, DW = 100000, 128
VC, DC = 96, 64
NF = 128
KW = 5
D = DW + NF
N = B * S
NW = 32
RPW = N // NW
TT = 256
NPOS = L - KW + 1


def _word_gather(idx_flat, table):
    mesh = plsc.VectorSubcoreMesh(core_axis_name="c", subcore_axis_name="s")

    @functools.partial(
        pl.kernel,
        out_type=jax.ShapeDtypeStruct((N, DW), jnp.float32),
        mesh=mesh,
        scratch_types=[
            pltpu.VMEM((RPW,), jnp.int32),
            pltpu.VMEM((RPW, DW), jnp.float32),
            pltpu.SemaphoreType.DMA,
        ],
    )
    def gk(idx_hbm, table_hbm, out_hbm, idx_v, rows_v, sem):
        wid = lax.axis_index("s") * 2 + lax.axis_index("c")
        base = wid * RPW
        pltpu.sync_copy(idx_hbm.at[pl.ds(base, RPW)], idx_v)
        pltpu.async_copy(table_hbm.at[idx_v], rows_v, sem).wait()
        pltpu.sync_copy(rows_v, out_hbm.at[pl.ds(base, RPW)])

    return gk(idx_flat, table)


def _tc_body(chars_ref, words_ref, ctab_ref, wcat_ref, bconv_ref,
             wg0_ref, bg0_ref, wt0_ref, bt0_ref,
             wg1_ref, bg1_ref, wt1_ref, bt1_ref,
             out_ref, m_ref):
    @pl.when(pl.program_id(0) == 0)
    def _():
        m_ref[...] = jnp.zeros((KW * NF, NF), jnp.bfloat16)
        mt = jnp.dot(ctab_ref[...], wcat_ref[...],
                     preferred_element_type=jnp.float32)
        for k in range(KW):
            m_ref[pl.ds(k * NF, VC), :] = (
                mt[:, k * NF:(k + 1) * NF].astype(jnp.bfloat16))

    chars = chars_ref[...]
    ci = lax.broadcasted_iota(jnp.int32, (TT, L, NF), 2)
    oh = jnp.where(ci == chars[:, :, None], 1.0, 0.0).astype(jnp.bfloat16)
    oh2 = oh.reshape(TT, L * NF)
    m = m_ref[...]

    acc = None
    for l in range(NPOS):
        p = lax.dot_general(oh2[:, l * NF:(l + KW) * NF], m,
                            (((1,), (0,)), ((), ())),
                            preferred_element_type=jnp.float32)
        acc = p if acc is None else jnp.maximum(acc, p)
    ce = jnp.maximum(acc + bconv_ref[...], 0.0)

    x = jnp.concatenate([words_ref[...], ce], axis=1)
    for wg, bg, wt, bt in ((wg0_ref, bg0_ref, wt0_ref, bt0_ref),
                           (wg1_ref, bg1_ref, wt1_ref, bt1_ref)):
        xb = x.astype(jnp.bfloat16)
        g = jax.nn.sigmoid(
            jnp.dot(xb, wg[...], preferred_element_type=jnp.float32) + bg[...])
        t = jnp.maximum(
            jnp.dot(xb, wt[...], preferred_element_type=jnp.float32) + bt[...],
            0.0)
        x = g * t + (1.0 - g) * x
    out_ref[...] = x


def _tc_fused(chars2, word_rows, ctab, wcat, bconv,
              wg0t, bg0, wt0t, bt0, wg1t, bg1, wt1t, bt1):
    const = lambda i: (0, 0)
    row = lambda i: (i, 0)
    return pl.pallas_call(
        _tc_body,
        grid=(N // TT,),
        in_specs=[
            pl.BlockSpec((TT, L), row),
            pl.BlockSpec((TT, DW), row),
            pl.BlockSpec((VC, DC), const),
            pl.BlockSpec((DC, KW * NF), const),
            pl.BlockSpec((1, NF), const),
            pl.BlockSpec((D, D), const),
            pl.BlockSpec((1, D), const),
            pl.BlockSpec((D, D), const),
            pl.BlockSpec((1, D), const),
            pl.BlockSpec((D, D), const),
            pl.BlockSpec((1, D), const),
            pl.BlockSpec((D, D), const),
            pl.BlockSpec((1, D), const),
        ],
        out_specs=pl.BlockSpec((TT, D), row),
        out_shape=jax.ShapeDtypeStruct((N, D), jnp.float32),
        scratch_shapes=[pltpu.VMEM((KW * NF, NF), jnp.bfloat16)],
    )(chars2, word_rows, ctab, wcat, bconv,
      wg0t, bg0, wt0t, bt0, wg1t, bg1, wt1t, bt1)


def kernel(word_idxs, char_idxs, word_table, char_table, W_conv, b_conv,
           Wt0, bt0, Wg0, bg0, Wt1, bt1, Wg1, bg1):
    widx = word_idxs.reshape(N).astype(jnp.int32)
    word_rows = _word_gather(widx, word_table)

    chars2 = char_idxs.reshape(N, L).astype(jnp.int32)
    wcat = jnp.transpose(W_conv[:, :, 0, :], (1, 2, 0)).reshape(DC, KW * NF)
    bconv = b_conv.reshape(1, NF)
    emb = _tc_fused(
        chars2, word_rows, char_table, wcat, bconv,
        Wg0.T.astype(jnp.bfloat16), bg0.reshape(1, D),
        Wt0.T.astype(jnp.bfloat16), bt0.reshape(1, D),
        Wg1.T.astype(jnp.bfloat16), bg1.reshape(1, D),
        Wt1.T.astype(jnp.bfloat16), bt1.reshape(1, D),
    )
    return emb.reshape(B, S, D)


# speedup vs baseline: 8.3564x; 1.1306x over previous
import functools

import numpy as np
import jax
import jax.numpy as jnp
from jax import lax
from jax.experimental import pallas as pl
from jax.experimental.pallas import tpu as pltpu
from jax.experimental.pallas import tpu_sc as plsc

B, S, L = 64, 400, 16
VW, DW = 100000, 128
VC, DC = 96, 64
NF = 128
KW = 5
D = DW + NF
N = B * S
NW = 32
RPW = N // NW
TT = 512
NPOS = L - KW + 1


def _word_gather(idx_flat, table):
    mesh = plsc.VectorSubcoreMesh(core_axis_name="c", subcore_axis_name="s")

    @functools.partial(
        pl.kernel,
        out_type=jax.ShapeDtypeStruct((N, DW), jnp.float32),
        mesh=mesh,
        scratch_types=[
            pltpu.VMEM((RPW,), jnp.int32),
            pltpu.VMEM((RPW, DW), jnp.float32),
            pltpu.SemaphoreType.DMA,
        ],
    )
    def gk(idx_hbm, table_hbm, out_hbm, idx_v, rows_v, sem):
        wid = lax.axis_index("s") * 2 + lax.axis_index("c")
        base = wid * RPW
        pltpu.sync_copy(idx_hbm.at[pl.ds(base, RPW)], idx_v)
        pltpu.async_copy(table_hbm.at[idx_v], rows_v, sem).wait()
        pltpu.sync_copy(rows_v, out_hbm.at[pl.ds(base, RPW)])

    return gk(idx_flat, table)


def _tc_body(chars_ref, words_ref, ctab_ref, wcat_ref, bconv_ref,
             exp_ref, cmod_ref,
             wg0_ref, bg0_ref, wt0_ref, bt0_ref,
             wg1_ref, bg1_ref, wt1_ref, bt1_ref,
             out_ref, m_ref):
    @pl.when(pl.program_id(0) == 0)
    def _():
        m_ref[...] = jnp.zeros((KW * NF, NF), jnp.bfloat16)
        mt = jnp.dot(ctab_ref[...], wcat_ref[...],
                     preferred_element_type=jnp.float32)
        for k in range(KW):
            m_ref[pl.ds(k * NF, VC), :] = (
                mt[:, k * NF:(k + 1) * NF].astype(jnp.bfloat16))

    chars_bf = chars_ref[...].astype(jnp.bfloat16)
    chars_rep = jnp.dot(chars_bf, exp_ref[...],
                        preferred_element_type=jnp.float32
                        ).astype(jnp.bfloat16)
    oh2 = jnp.where(chars_rep == cmod_ref[...],
                    jnp.bfloat16(1), jnp.bfloat16(0))
    m = m_ref[...]

    acc = None
    for l in range(NPOS):
        p = lax.dot_general(oh2[:, l * NF:(l + KW) * NF], m,
                            (((1,), (0,)), ((), ())),
                            preferred_element_type=jnp.float32)
        acc = p if acc is None else jnp.maximum(acc, p)
    ce = jnp.maximum(acc + bconv_ref[...], 0.0)

    x = jnp.concatenate([words_ref[...], ce], axis=1)
    for wg, bg, wt, bt in ((wg0_ref, bg0_ref, wt0_ref, bt0_ref),
                           (wg1_ref, bg1_ref, wt1_ref, bt1_ref)):
        xb = x.astype(jnp.bfloat16)
        g = jax.nn.sigmoid(
            jnp.dot(xb, wg[...], preferred_element_type=jnp.float32) + bg[...])
        t = jnp.maximum(
            jnp.dot(xb, wt[...], preferred_element_type=jnp.float32) + bt[...],
            0.0)
        x = g * t + (1.0 - g) * x
    out_ref[...] = x


def _tc_fused(chars2, word_rows, ctab, wcat, bconv, expander, cmod,
              wg0t, bg0, wt0t, bt0, wg1t, bg1, wt1t, bt1):
    const = lambda i: (0, 0)
    row = lambda i: (i, 0)
    return pl.pallas_call(
        _tc_body,
        grid=(N // TT,),
        in_specs=[
            pl.BlockSpec((TT, L), row),
            pl.BlockSpec((TT, DW), row),
            pl.BlockSpec((VC, DC), const),
            pl.BlockSpec((DC, KW * NF), const),
            pl.BlockSpec((1, NF), const),
            pl.BlockSpec((L, L * NF), const),
            pl.BlockSpec((1, L * NF), const),
            pl.BlockSpec((D, D), const),
            pl.BlockSpec((1, D), const),
            pl.BlockSpec((D, D), const),
            pl.BlockSpec((1, D), const),
            pl.BlockSpec((D, D), const),
            pl.BlockSpec((1, D), const),
            pl.BlockSpec((D, D), const),
            pl.BlockSpec((1, D), const),
        ],
        out_specs=pl.BlockSpec((TT, D), row),
        out_shape=jax.ShapeDtypeStruct((N, D), jnp.float32),
        scratch_shapes=[pltpu.VMEM((KW * NF, NF), jnp.bfloat16)],
    )(chars2, word_rows, ctab, wcat, bconv, expander, cmod,
      wg0t, bg0, wt0t, bt0, wg1t, bg1, wt1t, bt1)


def kernel(word_idxs, char_idxs, word_table, char_table, W_conv, b_conv,
           Wt0, bt0, Wg0, bg0, Wt1, bt1, Wg1, bg1):
    widx = word_idxs.reshape(N).astype(jnp.int32)
    word_rows = _word_gather(widx, word_table)

    chars2 = char_idxs.reshape(N, L).astype(jnp.int32)
    wcat = jnp.transpose(W_conv[:, :, 0, :], (1, 2, 0)).reshape(DC, KW * NF)
    bconv = b_conv.reshape(1, NF)
    cols = np.arange(L * NF)
    expander = jnp.asarray(
        (cols // NF == np.arange(L)[:, None]).astype(np.float32),
        dtype=jnp.bfloat16)
    cmod = jnp.asarray((cols % NF).astype(np.float32)[None, :],
                       dtype=jnp.bfloat16)
    emb = _tc_fused(
        chars2, word_rows, char_table, wcat, bconv, expander, cmod,
        Wg0.T.astype(jnp.bfloat16), bg0.reshape(1, D),
        Wt0.T.astype(jnp.bfloat16), bt0.reshape(1, D),
        Wg1.T.astype(jnp.bfloat16), bg1.reshape(1, D),
        Wt1.T.astype(jnp.bfloat16), bt1.reshape(1, D),
    )
    return emb.reshape(B, S, D)


# speedup vs baseline: 10.7976x; 1.2921x over previous
import functools

import numpy as np
import jax
import jax.numpy as jnp
from jax import lax
from jax.experimental import pallas as pl
from jax.experimental.pallas import tpu as pltpu
from jax.experimental.pallas import tpu_sc as plsc

B, S, L = 64, 400, 16
VW, DW = 100000, 128
VC, DC = 96, 64
NF = 128
KW = 5
D = DW + NF
N = B * S
NW = 32
RPW = N // NW
TT = 512
NPOS = L - KW + 1


def _word_gather(idx_flat, table):
    mesh = plsc.VectorSubcoreMesh(core_axis_name="c", subcore_axis_name="s")

    @functools.partial(
        pl.kernel,
        out_type=jax.ShapeDtypeStruct((N, DW), jnp.float32),
        mesh=mesh,
        scratch_types=[
            pltpu.VMEM((RPW,), jnp.int32),
            pltpu.VMEM((RPW, DW), jnp.float32),
            pltpu.SemaphoreType.DMA,
        ],
    )
    def gk(idx_hbm, table_hbm, out_hbm, idx_v, rows_v, sem):
        wid = lax.axis_index("s") * 2 + lax.axis_index("c")
        base = wid * RPW
        pltpu.sync_copy(idx_hbm.at[pl.ds(base, RPW)], idx_v)
        pltpu.async_copy(table_hbm.at[idx_v], rows_v, sem).wait()
        pltpu.sync_copy(rows_v, out_hbm.at[pl.ds(base, RPW)])

    return gk(idx_flat, table)


def _tc_body(chars_ref, words_ref, ctab_ref, wcat_ref, bconv_ref,
             exp_ref, cmod_ref,
             wg0_ref, bg0_ref, wt0_ref, bt0_ref,
             wg1_ref, bg1_ref, wt1_ref, bt1_ref,
             out_ref, m_ref):
    @pl.when(pl.program_id(0) == 0)
    def _():
        m_ref[...] = jnp.zeros(((KW + 1) * NF, 2 * NF), jnp.bfloat16)
        mt = jnp.dot(ctab_ref[...], wcat_ref[...],
                     preferred_element_type=jnp.float32)
        for k in range(KW):
            blk = mt[:, k * NF:(k + 1) * NF].astype(jnp.bfloat16)
            m_ref[pl.ds(k * NF, VC), 0:NF] = blk
            m_ref[pl.ds((k + 1) * NF, VC), NF:2 * NF] = blk

    chars_bf = chars_ref[...].astype(jnp.bfloat16)
    chars_rep = jnp.dot(chars_bf, exp_ref[...],
                        preferred_element_type=jnp.float32
                        ).astype(jnp.bfloat16)
    oh2 = jnp.where(chars_rep == cmod_ref[...],
                    jnp.bfloat16(1), jnp.bfloat16(0))
    m = m_ref[...]

    acc = None
    for p in range(NPOS // 2):
        pr = lax.dot_general(oh2[:, 2 * p * NF:(2 * p + KW + 1) * NF], m,
                             (((1,), (0,)), ((), ())),
                             preferred_element_type=jnp.float32)
        acc = pr if acc is None else jnp.maximum(acc, pr)
    acc = jnp.maximum(acc[:, :NF], acc[:, NF:])
    ce = jnp.maximum(acc + bconv_ref[...], 0.0)

    x = jnp.concatenate([words_ref[...], ce], axis=1)
    for wg, bg, wt, bt in ((wg0_ref, bg0_ref, wt0_ref, bt0_ref),
                           (wg1_ref, bg1_ref, wt1_ref, bt1_ref)):
        xb = x.astype(jnp.bfloat16)
        g = jax.nn.sigmoid(
            jnp.dot(xb, wg[...], preferred_element_type=jnp.float32) + bg[...])
        t = jnp.maximum(
            jnp.dot(xb, wt[...], preferred_element_type=jnp.float32) + bt[...],
            0.0)
        x = g * t + (1.0 - g) * x
    out_ref[...] = x


def _tc_fused(chars2, word_rows, ctab, wcat, bconv, expander, cmod,
              wg0t, bg0, wt0t, bt0, wg1t, bg1, wt1t, bt1):
    const = lambda i: (0, 0)
    row = lambda i: (i, 0)
    return pl.pallas_call(
        _tc_body,
        grid=(N // TT,),
        in_specs=[
            pl.BlockSpec((TT, L), row),
            pl.BlockSpec((TT, DW), row),
            pl.BlockSpec((VC, DC), const),
            pl.BlockSpec((DC, KW * NF), const),
            pl.BlockSpec((1, NF), const),
            pl.BlockSpec((L, L * NF), const),
            pl.BlockSpec((1, L * NF), const),
            pl.BlockSpec((D, D), const),
            pl.BlockSpec((1, D), const),
            pl.BlockSpec((D, D), const),
            pl.BlockSpec((1, D), const),
            pl.BlockSpec((D, D), const),
            pl.BlockSpec((1, D), const),
            pl.BlockSpec((D, D), const),
            pl.BlockSpec((1, D), const),
        ],
        out_specs=pl.BlockSpec((TT, D), row),
        out_shape=jax.ShapeDtypeStruct((N, D), jnp.float32),
        scratch_shapes=[pltpu.VMEM(((KW + 1) * NF, 2 * NF), jnp.bfloat16)],
    )(chars2, word_rows, ctab, wcat, bconv, expander, cmod,
      wg0t, bg0, wt0t, bt0, wg1t, bg1, wt1t, bt1)


def kernel(word_idxs, char_idxs, word_table, char_table, W_conv, b_conv,
           Wt0, bt0, Wg0, bg0, Wt1, bt1, Wg1, bg1):
    widx = word_idxs.reshape(N).astype(jnp.int32)
    word_rows = _word_gather(widx, word_table)

    chars2 = char_idxs.reshape(N, L).astype(jnp.int32)
    wcat = jnp.transpose(W_conv[:, :, 0, :], (1, 2, 0)).reshape(DC, KW * NF)
    bconv = b_conv.reshape(1, NF)
    cols = np.arange(L * NF)
    expander = jnp.asarray(
        (cols // NF == np.arange(L)[:, None]).astype(np.float32),
        dtype=jnp.bfloat16)
    cmod = jnp.asarray((cols % NF).astype(np.float32)[None, :],
                       dtype=jnp.bfloat16)
    emb = _tc_fused(
        chars2, word_rows, char_table, wcat, bconv, expander, cmod,
        Wg0.T.astype(jnp.bfloat16), bg0.reshape(1, D),
        Wt0.T.astype(jnp.bfloat16), bt0.reshape(1, D),
        Wg1.T.astype(jnp.bfloat16), bg1.reshape(1, D),
        Wt1.T.astype(jnp.bfloat16), bt1.reshape(1, D),
    )
    return emb.reshape(B, S, D)
